# baseline (speedup 1.0000x reference)
import jax
import jax.numpy as jnp
from jax import lax
from jax.experimental import pallas as pl
from jax.experimental.pallas import tpu as pltpu

NUM_CLASSES_K = 128
BATCH = 64
PIX = 32 * 32

NCHUNK = 16
CB = BATCH // NCHUNK
NBUF = 8


def _onehot_body(x_ref, o_ref, buf_ref, sem_ref):
    cls = lax.broadcasted_iota(jnp.int32, (CB, NUM_CLASSES_K, PIX), 1)
    for i in range(NCHUNK):
        k = i % NBUF
        if i >= NBUF:
            pltpu.make_async_copy(
                buf_ref.at[(i - NBUF) % NBUF],
                o_ref.at[pl.ds((i - NBUF) * CB, CB)],
                sem_ref.at[(i - NBUF) % NBUF],
            ).wait()
        xc = x_ref[pl.ds(i * CB, CB), :]
        buf_ref[k] = (xc[:, None, :] == cls).astype(jnp.float32)
        pltpu.make_async_copy(
            buf_ref.at[k],
            o_ref.at[pl.ds(i * CB, CB)],
            sem_ref.at[k],
        ).start()
    for i in range(max(NCHUNK - NBUF, 0), NCHUNK):
        pltpu.make_async_copy(
            buf_ref.at[i % NBUF],
            o_ref.at[pl.ds(i * CB, CB)],
            sem_ref.at[i % NBUF],
        ).wait()


def kernel(x):
    x = x.astype(jnp.int32).reshape(BATCH, PIX)
    out = pl.pallas_call(
        _onehot_body,
        in_specs=[pl.BlockSpec(memory_space=pltpu.MemorySpace.VMEM)],
        out_specs=pl.BlockSpec(memory_space=pltpu.MemorySpace.HBM),
        out_shape=jax.ShapeDtypeStruct((BATCH, NUM_CLASSES_K, PIX), jnp.float32),
        scratch_shapes=[
            pltpu.MemorySpace.VMEM((NBUF, CB, NUM_CLASSES_K, PIX), jnp.float32),
            pltpu.SemaphoreType.DMA((NBUF,)),
        ],
    )(x)
    return out.reshape(BATCH, NUM_CLASSES_K, 32, 32)
